# baseline (speedup 1.0000x reference)
import functools

import jax
import jax.numpy as jnp
from jax import lax
from jax.experimental import pallas as pl
from jax.experimental.pallas import tpu as pltpu
from jax.experimental.pallas import tpu_sc as plsc

NUM_CORES = 2
NUM_SUBCORES = 16
NW = NUM_CORES * NUM_SUBCORES

BATCH, SEQ = 4096, 50
DIM = 728
VOCAB = 10

BBLK = 256
NBBLK = BATCH // BBLK
SBLK = SEQ // 2
JC = 7
NCH = (DIM // 8) // JC
NQ = SBLK * NCH
NG = BBLK // 16


@functools.partial(
    pl.kernel,
    out_type=jax.ShapeDtypeStruct((SEQ, DIM, BATCH), jnp.float32),
    mesh=plsc.VectorSubcoreMesh(core_axis_name="c", subcore_axis_name="s"),
    scratch_types=[
        pltpu.VMEM((SBLK, BBLK), jnp.int32),
        pltpu.VMEM((DIM * 16,), jnp.float32),
        pltpu.VMEM((JC * 8, BBLK), jnp.float32),
        pltpu.VMEM((JC * 8, BBLK), jnp.float32),
        pltpu.SemaphoreType.DMA,
        pltpu.SemaphoreType.DMA,
    ],
    compiler_params=pltpu.CompilerParams(needs_layout_passes=False),
)
def _embedding_lookup(idx_hbm, table_hbm, out_hbm,
                      idx_v, table_v, buf0, buf1, wsem0, wsem1):
    wid = lax.axis_index("s") * NUM_CORES + lax.axis_index("c")
    k = wid % NBBLK
    par = wid // NBBLK
    pltpu.sync_copy(idx_hbm.at[k, par], idx_v)
    pltpu.sync_copy(table_hbm, table_v)

    bufs = (buf0, buf1)
    wsems = (wsem0, wsem1)

    def dst_ref(q):
        m = q // NCH
        c = q % NCH
        s = par + 2 * m
        return out_hbm.at[s, pl.ds(c * (JC * 8), JC * 8),
                          pl.ds(k * BBLK, BBLK)]

    @pl.loop(0, NQ + 1, step=2)
    def _(qq):
        for t in range(2):
            q = qq + t

            @pl.when(q < NQ)
            def _():
                m = q // NCH
                c = q % NCH
                buf = bufs[t]

                @pl.when(q >= 2)
                def _():
                    pltpu.make_async_copy(buf, dst_ref(q - 2), wsems[t]).wait()

                idxs = [idx_v[m, pl.ds(g * 16, 16)] for g in range(NG)]

                for jl in range(JC):
                    dbase = (c * JC + jl) * 8
                    for dd in range(8):
                        d16 = (dbase + dd) * 16
                        for g in range(NG):
                            buf[jl * 8 + dd, pl.ds(g * 16, 16)] = (
                                plsc.load_gather(table_v, [idxs[g] + d16]))

                pltpu.async_copy(buf, dst_ref(q), wsems[t])

    pltpu.make_async_copy(buf0, dst_ref(NQ - 2), wsem0).wait()
    pltpu.make_async_copy(buf1, dst_ref(NQ - 1), wsem1).wait()


def kernel(inputs, table):
    idx = jnp.asarray(inputs, jnp.int32)
    idx4 = (idx.T.reshape(SBLK, 2, NBBLK, BBLK)
            .transpose(2, 1, 0, 3))
    table_t = jnp.pad(table.T, ((0, 0), (0, 16 - VOCAB)))
    out_t = _embedding_lookup(idx4, table_t.reshape(-1))
    return out_t.transpose(2, 0, 1)


# speedup vs baseline: 3.6238x; 3.6238x over previous
import functools

import jax
import jax.numpy as jnp
from jax import lax
from jax.experimental import pallas as pl
from jax.experimental.pallas import tpu as pltpu
from jax.experimental.pallas import tpu_sc as plsc

NUM_CORES = 2
NUM_SUBCORES = 16
NW = NUM_CORES * NUM_SUBCORES

BATCH, SEQ = 4096, 50
DIM = 728
VOCAB = 10

BBLK = 256
NBBLK = BATCH // BBLK
SBLK = SEQ // 2
JC = 7
NCH = (DIM // 8) // JC
NQ = SBLK * NCH
NG = BBLK // 16


@functools.partial(
    pl.kernel,
    out_type=jax.ShapeDtypeStruct((SEQ, DIM, BATCH), jnp.float32),
    mesh=plsc.VectorSubcoreMesh(core_axis_name="c", subcore_axis_name="s"),
    scratch_types=[
        pltpu.VMEM((SBLK, BBLK), jnp.int32),
        pltpu.VMEM((DIM, 16), jnp.float32),
        pltpu.VMEM((JC * 8, BBLK), jnp.float32),
        pltpu.VMEM((JC * 8, BBLK), jnp.float32),
        pltpu.SemaphoreType.DMA,
        pltpu.SemaphoreType.DMA,
    ],
    compiler_params=pltpu.CompilerParams(needs_layout_passes=False),
)
def _embedding_lookup(idx_hbm, table_hbm, out_hbm,
                      idx_v, table_v, buf0, buf1, wsem0, wsem1):
    wid = lax.axis_index("s") * NUM_CORES + lax.axis_index("c")
    k = wid % NBBLK
    par = wid // NBBLK
    pltpu.sync_copy(idx_hbm.at[k, par], idx_v)
    pltpu.sync_copy(table_hbm, table_v)

    bufs = (buf0, buf1)
    wsems = (wsem0, wsem1)

    def dst_ref(q):
        m = q // NCH
        c = q % NCH
        s = par + 2 * m
        return out_hbm.at[s, pl.ds(c * (JC * 8), JC * 8),
                          pl.ds(k * BBLK, BBLK)]

    @pl.loop(0, NQ + 1, step=2)
    def _(qq):
        for t in range(2):
            q = qq + t

            @pl.when(q < NQ)
            def _():
                m = q // NCH
                c = q % NCH
                buf = bufs[t]

                @pl.when(q >= 2)
                def _():
                    pltpu.make_async_copy(buf, dst_ref(q - 2), wsems[t]).wait()

                idxs = [idx_v[m, pl.ds(g * 16, 16)] for g in range(NG)]

                for jl in range(JC):
                    dbase = (c * JC + jl) * 8
                    tds = [table_v[dbase + dd, :] for dd in range(8)]
                    for dd in range(8):
                        for g in range(NG):
                            buf[jl * 8 + dd, pl.ds(g * 16, 16)] = (
                                tds[dd].at[idxs[g]].get(
                                    mode="promise_in_bounds"))

                pltpu.async_copy(buf, dst_ref(q), wsems[t])

    pltpu.make_async_copy(buf0, dst_ref(NQ - 2), wsem0).wait()
    pltpu.make_async_copy(buf1, dst_ref(NQ - 1), wsem1).wait()


def kernel(inputs, table):
    idx = jnp.asarray(inputs, jnp.int32)
    idx4 = (idx.T.reshape(SBLK, 2, NBBLK, BBLK)
            .transpose(2, 1, 0, 3))
    table_t = jnp.pad(table.T, ((0, 0), (0, 16 - VOCAB)))
    out_t = _embedding_lookup(idx4, table_t)
    return out_t.transpose(2, 0, 1)
